# baseline (speedup 1.0000x reference)
import jax
import jax.numpy as jnp
from jax import lax
from jax.experimental import pallas as pl
from jax.experimental.pallas import tpu as pltpu
from jax.experimental.pallas import tpu_sc as plsc

NC = 2
NS = 16
L = 16
NW = NC * NS

E = 320000
D = 128
EPW = E // NW
CH = 128
NFULL = EPW // CH
TAIL_OFF = EPW - CH
NCHUNK = NFULL + 1
NPAIR = NFULL // 2
SW = 129


def _body(x_hbm, src_hbm, dst_hbm, out_hbm,
          sidx_v, didx_v, out_v, sr0, sr1, dr0, dr1, tr_v,
          sem_s0, sem_d0, sem_s1, sem_d1):
  wid = lax.axis_index("s") * NC + lax.axis_index("c")
  base = wid * EPW
  rows0 = lax.broadcasted_iota(jnp.int32, (L,), 0)

  pltpu.sync_copy(src_hbm.at[pl.ds(base, EPW)], sidx_v)
  pltpu.sync_copy(dst_hbm.at[pl.ds(base, EPW)], didx_v)

  def fire(off, srows, drows, sem_s, sem_d):
    pltpu.async_copy(x_hbm.at[sidx_v.at[pl.ds(off, CH)]], srows, sem_s)
    pltpu.async_copy(x_hbm.at[didx_v.at[pl.ds(off, CH)]], drows, sem_d)

  def wait(srows, drows, sem_s, sem_d):
    pltpu.make_async_copy(x_hbm.at[sidx_v.at[pl.ds(0, CH)]], srows, sem_s).wait()
    pltpu.make_async_copy(x_hbm.at[didx_v.at[pl.ds(0, CH)]], drows, sem_d).wait()

  colbase = rows0 * SW

  def compute(off, srows, drows):
    def edge_body(e, carry):
      acc = jnp.zeros((L,), jnp.float32)
      for k in range(D // L):
        s = srows[e, pl.ds(k * L, L)]
        d = drows[e, pl.ds(k * L, L)]
        acc = acc + s * d
      plsc.store_scatter(tr_v, [colbase + e], acc)
      return carry
    lax.fori_loop(0, CH, edge_body, 0, unroll=False)

    for cg in range(CH // L):
      tot = jnp.zeros((L,), jnp.float32)
      for k in range(L):
        tot = tot + plsc.load_gather(tr_v, [jnp.full((L,), k * SW + cg * L, jnp.int32) + rows0])
      out_v[pl.ds(off + cg * L, L)] = tot

  fire(0, sr0, dr0, sem_s0, sem_d0)

  def pair_body(t, carry):
    j0 = 2 * t
    fire((j0 + 1) * CH, sr1, dr1, sem_s1, sem_d1)
    wait(sr0, dr0, sem_s0, sem_d0)
    compute(j0 * CH, sr0, dr0)
    off2 = jnp.minimum((j0 + 2) * CH, TAIL_OFF)
    fire(off2, sr0, dr0, sem_s0, sem_d0)
    wait(sr1, dr1, sem_s1, sem_d1)
    compute((j0 + 1) * CH, sr1, dr1)
    return carry

  lax.fori_loop(0, NPAIR, pair_body, 0, unroll=False)

  wait(sr0, dr0, sem_s0, sem_d0)
  compute(TAIL_OFF, sr0, dr0)

  pltpu.sync_copy(out_v, out_hbm.at[pl.ds(base, EPW)])


@jax.jit
def kernel(x, edge_index):
  mesh = plsc.VectorSubcoreMesh(core_axis_name="c", subcore_axis_name="s")
  k = pl.kernel(
      _body,
      out_type=jax.ShapeDtypeStruct((E,), jnp.float32),
      mesh=mesh,
      compiler_params=pltpu.CompilerParams(needs_layout_passes=False),
      scratch_types=[
          pltpu.VMEM((EPW,), jnp.int32),
          pltpu.VMEM((EPW,), jnp.int32),
          pltpu.VMEM((EPW,), jnp.float32),
          pltpu.VMEM((CH, D), jnp.float32),
          pltpu.VMEM((CH, D), jnp.float32),
          pltpu.VMEM((CH, D), jnp.float32),
          pltpu.VMEM((CH, D), jnp.float32),
          pltpu.VMEM((L * SW,), jnp.float32),
          pltpu.SemaphoreType.DMA,
          pltpu.SemaphoreType.DMA,
          pltpu.SemaphoreType.DMA,
          pltpu.SemaphoreType.DMA,
      ],
  )
  return k(x, edge_index[0], edge_index[1])
